# Initial kernel scaffold; baseline (speedup 1.0000x reference)
import jax
import jax.numpy as jnp
from jax.experimental import pallas as pl


def kernel(x, edge_index, nodes, W_enc, W_cls):
    raise NotImplementedError("write your pallas kernel here")


# speedup vs baseline: 6.6784x; 6.6784x over previous
import functools

import jax
import jax.numpy as jnp
from jax import lax
from jax.experimental import pallas as pl
from jax.experimental.pallas import tpu as pltpu
from jax.experimental.pallas import tpu_sc as plsc

N_NODES = 10000
N_EDGES = 320000
D_FEAT = 128
D_AUG = 144
NC = 2
NS = 16
NW = NC * NS
EPW = N_EDGES // NW
CH = 80
NCH = EPW // CH
NG = 5
NCH_G = NCH // NG
N_PAD = 10240
RPS = N_PAD // NS
ZCH = 80
B_PAD = 10240
GCH = 80
GNCH = B_PAD // NW // GCH

_sc_mesh = plsc.VectorSubcoreMesh(core_axis_name="c", subcore_axis_name="s")


@functools.partial(
    pl.kernel,
    out_type=jax.ShapeDtypeStruct((NC, N_PAD, D_AUG), jnp.float32),
    mesh=_sc_mesh,
    compiler_params=pltpu.CompilerParams(use_tc_tiling_on_sc=False),
    scratch_types=[
        pltpu.VMEM((NCH_G, CH), jnp.int32),
        pltpu.VMEM((NCH_G, CH), jnp.int32),
        pltpu.VMEM((CH, D_AUG), jnp.float32),
        pltpu.VMEM_SHARED((N_PAD, D_AUG), jnp.float32),
        pltpu.SemaphoreType.DMA,
    ],
)
def _sc_segment_sum(xa_hbm, src_hbm, dst_hbm, zeros_hbm, out_hbm,
                    src_v, dst_v, rows_v, acc_sh, sem):
    c = lax.axis_index("c")
    s = lax.axis_index("s")
    wid = s * NC + c

    pltpu.sync_copy(zeros_hbm, rows_v)
    for k in range(RPS // ZCH):
        pltpu.sync_copy(rows_v, acc_sh.at[pl.ds(s * RPS + k * ZCH, ZCH)])
    plsc.subcore_barrier()

    def body(j, carry):
        pltpu.async_copy(xa_hbm.at[src_v.at[j]], rows_v, sem).wait()
        pltpu.sync_copy(rows_v, acc_sh.at[dst_v.at[j]], add=True)
        return carry

    for g in range(NG):
        pltpu.sync_copy(src_hbm.at[wid, pl.ds(g * NCH_G, NCH_G)], src_v)
        pltpu.sync_copy(dst_hbm.at[wid, pl.ds(g * NCH_G, NCH_G)], dst_v)
        lax.fori_loop(0, NCH_G, body, 0)
    plsc.subcore_barrier()

    for k in range(RPS // ZCH):
        pltpu.sync_copy(acc_sh.at[pl.ds(s * RPS + k * ZCH, ZCH)], rows_v)
        pltpu.sync_copy(rows_v, out_hbm.at[c, pl.ds(s * RPS + k * ZCH, ZCH)])


@functools.partial(
    pl.kernel,
    out_type=jax.ShapeDtypeStruct((B_PAD, D_FEAT), jnp.float32),
    mesh=_sc_mesh,
    compiler_params=pltpu.CompilerParams(use_tc_tiling_on_sc=False),
    scratch_types=[
        pltpu.VMEM((GNCH, GCH), jnp.int32),
        pltpu.VMEM((GCH, D_FEAT), jnp.float32),
        pltpu.SemaphoreType.DMA,
    ],
)
def _sc_gather_rows(scores_hbm, nodes_hbm, out_hbm, idx_v, rows_v, sem):
    c = lax.axis_index("c")
    s = lax.axis_index("s")
    wid = s * NC + c
    base = wid * (GNCH * GCH)
    pltpu.sync_copy(nodes_hbm.at[wid], idx_v)

    def body(j, carry):
        pltpu.async_copy(scores_hbm.at[idx_v.at[j]], rows_v, sem).wait()
        pltpu.sync_copy(rows_v, out_hbm.at[pl.ds(base + j * GCH, GCH)])
        return carry

    lax.fori_loop(0, GNCH, body, 0)


def _tc_dense_body(p0_ref, p1_ref, x_ref, wenc_ref, wcls_ref, o_ref):
    acc = p0_ref[...] + p1_ref[...]
    deg = jnp.clip(acc[:, D_FEAT:D_FEAT + 1], 1.0, None)
    neigh_mean = acc[:, :D_FEAT] / deg
    combined = jnp.concatenate([x_ref[...], neigh_mean], axis=1)
    h = lax.dot_general(combined, wenc_ref[...], (((1,), (1,)), ((), ())),
                        preferred_element_type=jnp.float32)
    h = jnp.maximum(h, 0.0)
    o_ref[...] = lax.dot_general(h, wcls_ref[...], (((1,), (1,)), ((), ())),
                                 preferred_element_type=jnp.float32)


_TC_BLK = 1000


def _tc_dense(p0, p1, x, W_enc, W_cls):
    grid = (N_NODES // _TC_BLK,)
    return pl.pallas_call(
        _tc_dense_body,
        grid=grid,
        in_specs=[
            pl.BlockSpec((_TC_BLK, D_AUG), lambda i: (i, 0)),
            pl.BlockSpec((_TC_BLK, D_AUG), lambda i: (i, 0)),
            pl.BlockSpec((_TC_BLK, D_FEAT), lambda i: (i, 0)),
            pl.BlockSpec(W_enc.shape, lambda i: (0, 0)),
            pl.BlockSpec(W_cls.shape, lambda i: (0, 0)),
        ],
        out_specs=pl.BlockSpec((_TC_BLK, W_cls.shape[0]), lambda i: (i, 0)),
        out_shape=jax.ShapeDtypeStruct((N_NODES, W_cls.shape[0]), jnp.float32),
    )(p0, p1, x, W_enc, W_cls)


@jax.jit
def kernel(x, edge_index, nodes, W_enc, W_cls):
    src = edge_index[0].astype(jnp.int32).reshape(NW, NCH, CH)
    dst = edge_index[1].astype(jnp.int32).reshape(NW, NCH, CH)
    xa = jnp.concatenate(
        [x, jnp.ones((N_NODES, 1), jnp.float32),
         jnp.zeros((N_NODES, D_AUG - D_FEAT - 1), jnp.float32)], axis=1)
    zeros = jnp.zeros((CH, D_AUG), jnp.float32)

    parts = _sc_segment_sum(xa, src, dst, zeros)
    scores = _tc_dense(parts[0], parts[1], x, W_enc, W_cls)

    nodes_p = jnp.concatenate(
        [nodes.astype(jnp.int32),
         jnp.zeros((B_PAD - nodes.shape[0],), jnp.int32)]).reshape(NW, GNCH, GCH)
    out = _sc_gather_rows(scores, nodes_p)
    return out[:nodes.shape[0]]


# speedup vs baseline: 8.9515x; 1.3404x over previous
import functools

import jax
import jax.numpy as jnp
from jax import lax
from jax.experimental import pallas as pl
from jax.experimental.pallas import tpu as pltpu
from jax.experimental.pallas import tpu_sc as plsc

N_NODES = 10000
N_EDGES = 320000
D_FEAT = 128
D_AUG = 144
NC = 2
NS = 16
NW = NC * NS
EPW = N_EDGES // NW
CH = 80
NCH = EPW // CH
NG = 5
NCH_G = NCH // NG
N_PAD = 10240
RPS = N_PAD // NS
ZCH = 80
B_PAD = 10240
GCH = 80
GNCH = B_PAD // NW // GCH

_sc_mesh = plsc.VectorSubcoreMesh(core_axis_name="c", subcore_axis_name="s")


@functools.partial(
    pl.kernel,
    out_type=jax.ShapeDtypeStruct((NC, N_PAD, D_AUG), jnp.float32),
    mesh=_sc_mesh,
    compiler_params=pltpu.CompilerParams(use_tc_tiling_on_sc=False),
    scratch_types=[
        pltpu.VMEM((NCH_G, CH), jnp.int32),
        pltpu.VMEM((NCH_G, CH), jnp.int32),
        pltpu.VMEM((CH, D_AUG), jnp.float32),
        pltpu.VMEM((CH, D_AUG), jnp.float32),
        pltpu.VMEM_SHARED((N_PAD, D_AUG), jnp.float32),
        pltpu.SemaphoreType.DMA,
        pltpu.SemaphoreType.DMA,
    ],
)
def _sc_segment_sum(xa_hbm, src_hbm, dst_hbm, zeros_hbm, out_hbm,
                    src_v, dst_v, rows_a, rows_b, acc_sh, sem_a, sem_b):
    c = lax.axis_index("c")
    s = lax.axis_index("s")
    wid = s * NC + c

    pltpu.sync_copy(zeros_hbm, rows_a)
    for k in range(RPS // ZCH):
        pltpu.sync_copy(rows_a, acc_sh.at[pl.ds(s * RPS + k * ZCH, ZCH)])
    plsc.subcore_barrier()

    def gather(j, buf, sem):
        pltpu.async_copy(xa_hbm.at[src_v.at[j]], buf, sem)

    def scatter(j, buf):
        pltpu.sync_copy(buf, acc_sh.at[dst_v.at[j]], add=True)

    def drain(buf, sem):
        pltpu.make_async_copy(xa_hbm.at[pl.ds(0, CH)], buf, sem).wait()

    def pair(k, carry):
        gather(2 * k + 1, rows_b, sem_b)
        drain(rows_a, sem_a)
        scatter(2 * k, rows_a)
        gather(2 * k + 2, rows_a, sem_a)
        drain(rows_b, sem_b)
        scatter(2 * k + 1, rows_b)
        return carry

    for g in range(NG):
        pltpu.sync_copy(src_hbm.at[wid, pl.ds(g * NCH_G, NCH_G)], src_v)
        pltpu.sync_copy(dst_hbm.at[wid, pl.ds(g * NCH_G, NCH_G)], dst_v)
        gather(0, rows_a, sem_a)
        lax.fori_loop(0, (NCH_G - 1) // 2, pair, 0)
        drain(rows_a, sem_a)
        scatter(NCH_G - 1, rows_a)
    plsc.subcore_barrier()

    for k in range(RPS // ZCH):
        pltpu.sync_copy(acc_sh.at[pl.ds(s * RPS + k * ZCH, ZCH)], rows_a)
        pltpu.sync_copy(rows_a, out_hbm.at[c, pl.ds(s * RPS + k * ZCH, ZCH)])


@functools.partial(
    pl.kernel,
    out_type=jax.ShapeDtypeStruct((B_PAD, D_FEAT), jnp.float32),
    mesh=_sc_mesh,
    compiler_params=pltpu.CompilerParams(use_tc_tiling_on_sc=False),
    scratch_types=[
        pltpu.VMEM((GNCH, GCH), jnp.int32),
        pltpu.VMEM((GCH, D_FEAT), jnp.float32),
        pltpu.SemaphoreType.DMA,
    ],
)
def _sc_gather_rows(scores_hbm, nodes_hbm, out_hbm, idx_v, rows_v, sem):
    c = lax.axis_index("c")
    s = lax.axis_index("s")
    wid = s * NC + c
    base = wid * (GNCH * GCH)
    pltpu.sync_copy(nodes_hbm.at[wid], idx_v)

    def body(j, carry):
        pltpu.async_copy(scores_hbm.at[idx_v.at[j]], rows_v, sem).wait()
        pltpu.sync_copy(rows_v, out_hbm.at[pl.ds(base + j * GCH, GCH)])
        return carry

    lax.fori_loop(0, GNCH, body, 0)


def _tc_dense_body(p0_ref, p1_ref, x_ref, wenc_ref, wcls_ref, o_ref):
    acc = p0_ref[...] + p1_ref[...]
    deg = jnp.clip(acc[:, D_FEAT:D_FEAT + 1], 1.0, None)
    neigh_mean = acc[:, :D_FEAT] / deg
    combined = jnp.concatenate([x_ref[...], neigh_mean], axis=1)
    h = lax.dot_general(combined, wenc_ref[...], (((1,), (1,)), ((), ())),
                        preferred_element_type=jnp.float32)
    h = jnp.maximum(h, 0.0)
    o_ref[...] = lax.dot_general(h, wcls_ref[...], (((1,), (1,)), ((), ())),
                                 preferred_element_type=jnp.float32)


_TC_BLK = 1000


def _tc_dense(p0, p1, x, W_enc, W_cls):
    grid = (N_NODES // _TC_BLK,)
    return pl.pallas_call(
        _tc_dense_body,
        grid=grid,
        in_specs=[
            pl.BlockSpec((_TC_BLK, D_AUG), lambda i: (i, 0)),
            pl.BlockSpec((_TC_BLK, D_AUG), lambda i: (i, 0)),
            pl.BlockSpec((_TC_BLK, D_FEAT), lambda i: (i, 0)),
            pl.BlockSpec(W_enc.shape, lambda i: (0, 0)),
            pl.BlockSpec(W_cls.shape, lambda i: (0, 0)),
        ],
        out_specs=pl.BlockSpec((_TC_BLK, W_cls.shape[0]), lambda i: (i, 0)),
        out_shape=jax.ShapeDtypeStruct((N_NODES, W_cls.shape[0]), jnp.float32),
    )(p0, p1, x, W_enc, W_cls)


@jax.jit
def kernel(x, edge_index, nodes, W_enc, W_cls):
    src = edge_index[0].astype(jnp.int32).reshape(NW, NCH, CH)
    dst = edge_index[1].astype(jnp.int32).reshape(NW, NCH, CH)
    xa = jnp.concatenate(
        [x, jnp.ones((N_NODES, 1), jnp.float32),
         jnp.zeros((N_NODES, D_AUG - D_FEAT - 1), jnp.float32)], axis=1)
    zeros = jnp.zeros((CH, D_AUG), jnp.float32)

    parts = _sc_segment_sum(xa, src, dst, zeros)
    scores = _tc_dense(parts[0], parts[1], x, W_enc, W_cls)

    nodes_p = jnp.concatenate(
        [nodes.astype(jnp.int32),
         jnp.zeros((B_PAD - nodes.shape[0],), jnp.int32)]).reshape(NW, GNCH, GCH)
    out = _sc_gather_rows(scores, nodes_p)
    return out[:nodes.shape[0]]
